# baseline (speedup 1.0000x reference)
import dataclasses
import functools

import jax
import jax.numpy as jnp
from jax import lax
from jax.experimental import pallas as pl
from jax.experimental.pallas import tpu as pltpu
from jax.experimental.pallas import tpu_sc as plsc

_BATCH = 4096
_HIST = 200
_BUCKET_LEN = 12
_OUT_DIM = 72
_NTHR = 8
_CW = 512
_NCH = _BATCH // _CW
_NW = 32
_HMAX = (_HIST + _NW - 1) // _NW
_REP = 1025


def _ids_chunk(ts_v, thr_v, id_v, base):
    @pl.loop(0, _CW, step=16)
    def _(v0):
        tsv = ts_v[pl.ds(base + v0, 16)]
        acc = jnp.zeros((16,), jnp.int32)
        for i in range(_NTHR):
            acc = acc + jnp.where(tsv >= thr_v[i, :], 1, 0).astype(jnp.int32)
        id_v[pl.ds(base + v0, 16)] = acc


def _fill_stage(tab_v, id_v, st, base):
    @plsc.parallel_loop(0, _CW, 16, unroll=16)
    def _(v0):
        idb = id_v[pl.ds(base + v0, 16)]
        depth = 16
        pend = []
        for c in range(_OUT_DIM):
            pend.append(
                (c, plsc.load_gather(tab_v.at[pl.ds(c * 16, 16)], [idb])))
            if len(pend) >= depth:
                cc, val = pend.pop(0)
                st[cc, pl.ds(v0, 16)] = val
        for cc, val in pend:
            st[cc, pl.ds(v0, 16)] = val


def _sc_body(tab_hbm, thr_hbm, ts_hbm, out_hbm,
             tab_v, thr_v, ts_v, id_v, st_a, st_b,
             sem_i, sem_a, sem_b):
    wid = lax.axis_index("s") * 2 + lax.axis_index("c")
    pltpu.sync_copy(tab_hbm, tab_v)
    pltpu.sync_copy(thr_hbm, thr_v)

    @pl.loop(0, _HMAX)
    def _(k):
        h = wid + _NW * k

        @pl.when(h < _HIST)
        def _():
            pltpu.async_copy(ts_hbm.at[h], ts_v, sem_i).wait()
            @pl.loop(0, _BATCH, step=16)
            def _(v0):
                tsv = ts_v[pl.ds(v0, 16)]
                acc = jnp.zeros((16,), jnp.int32)
                for i in range(_NTHR):
                    acc = acc + jnp.where(tsv >= thr_v[i, :], 1, 0
                                          ).astype(jnp.int32)
                id_v[pl.ds(v0, 16)] = acc

            @pl.loop(0, _NCH // 2)
            def _(j):
                ch0 = 2 * j

                @pl.when(j > 0)
                def _():
                    pltpu.make_async_copy(
                        st_a, out_hbm.at[h, :, pl.ds(0, _CW)], sem_a).wait()
                _fill_stage(tab_v, id_v, st_a, ch0 * _CW)
                pltpu.async_copy(
                    st_a, out_hbm.at[h, :, pl.ds(ch0 * _CW, _CW)], sem_a)

                @pl.when(j > 0)
                def _():
                    pltpu.make_async_copy(
                        st_b, out_hbm.at[h, :, pl.ds(0, _CW)], sem_b).wait()
                _fill_stage(tab_v, id_v, st_b, (ch0 + 1) * _CW)
                pltpu.async_copy(
                    st_b, out_hbm.at[h, :, pl.ds((ch0 + 1) * _CW, _CW)], sem_b)

            pltpu.make_async_copy(
                st_a, out_hbm.at[h, :, pl.ds(0, _CW)], sem_a).wait()
            pltpu.make_async_copy(
                st_b, out_hbm.at[h, :, pl.ds(0, _CW)], sem_b).wait()


def kernel(timestamps, time_emb_weight):
    boundaries = jnp.concatenate(
        [jnp.zeros((1,), jnp.float32),
         jnp.exp(jnp.arange(_BUCKET_LEN, dtype=jnp.float32))], axis=0)
    rel = boundaries[:_NTHR]
    m0 = jnp.floor(rel * 3600000.0).astype(jnp.int32)
    cand = m0[:, None] + jnp.arange(-1024, 1025, dtype=jnp.int32)[None, :]
    ok = cand.astype(jnp.float32) / 3600000.0 > rel[:, None]
    m = jnp.min(jnp.where(ok, cand, jnp.int32(2**31 - 1)), axis=1)
    thr = jnp.broadcast_to(m[:, None], (_NTHR, 16))

    tab_flat = jnp.zeros((_OUT_DIM, 16), jnp.float32).at[:, :14].set(
        time_emb_weight.T).reshape(-1)
    ts_t = timestamps.T

    mesh = plsc.VectorSubcoreMesh(core_axis_name="c", subcore_axis_name="s")
    cp = pltpu.CompilerParams()
    if "needs_layout_passes" in pltpu.CompilerParams.__dataclass_fields__:
        cp = dataclasses.replace(cp, needs_layout_passes=False)
    run = pl.kernel(
        _sc_body,
        out_type=jax.ShapeDtypeStruct((_HIST, _OUT_DIM, _BATCH), jnp.float32),
        mesh=mesh,
        compiler_params=cp,
        scratch_types=[
            pltpu.VMEM((_OUT_DIM * 16,), jnp.float32),
            pltpu.VMEM((_NTHR, 16), jnp.int32),
            pltpu.VMEM((_BATCH,), jnp.int32),
            pltpu.VMEM((_BATCH,), jnp.int32),
            pltpu.VMEM((_OUT_DIM, _CW), jnp.float32),
            pltpu.VMEM((_OUT_DIM, _CW), jnp.float32),
            pltpu.SemaphoreType.DMA,
            pltpu.SemaphoreType.DMA,
            pltpu.SemaphoreType.DMA,
        ],
    )
    out_t = run(tab_flat, thr, ts_t)
    return jnp.transpose(out_t, (2, 0, 1))


# speedup vs baseline: 1.8351x; 1.8351x over previous
import dataclasses
import functools

import jax
import jax.numpy as jnp
from jax import lax
from jax.experimental import pallas as pl
from jax.experimental.pallas import tpu as pltpu
from jax.experimental.pallas import tpu_sc as plsc

_BATCH = 4096
_HIST = 200
_BUCKET_LEN = 12
_OUT_DIM = 72
_NTHR = 8
_CW = 512
_NCH = _BATCH // _CW
_NW = 32
_HMAX = (_HIST + _NW - 1) // _NW
_REP = 1025


def _ids_chunk(ts_v, thr_v, id_v, base):
    @pl.loop(0, _CW, step=16)
    def _(v0):
        tsv = ts_v[pl.ds(base + v0, 16)]
        acc = jnp.zeros((16,), jnp.int32)
        for i in range(_NTHR):
            acc = acc + jnp.where(tsv >= thr_v[i, :], 1, 0).astype(jnp.int32)
        id_v[pl.ds(base + v0, 16)] = acc


def _fill_stage(tab_v, id_v, st, base):
    @plsc.parallel_loop(0, _CW, 16, unroll=8)
    def _(v0):
        idb = id_v[pl.ds(base + v0, 16)]
        depth = 4
        pend = []
        for c in range(_OUT_DIM):
            pend.append(
                (c, plsc.load_gather(tab_v.at[pl.ds(c * 16, 16)], [idb])))
            if len(pend) >= depth:
                cc, val = pend.pop(0)
                st[cc, pl.ds(v0, 16)] = val
        for cc, val in pend:
            st[cc, pl.ds(v0, 16)] = val


def _sc_body(tab_hbm, thr_hbm, ts_hbm, out_hbm,
             tab_v, thr_v, ts_v, id_v, st_a, st_b,
             sem_i, sem_a, sem_b):
    wid = lax.axis_index("s") * 2 + lax.axis_index("c")
    pltpu.sync_copy(tab_hbm, tab_v)
    pltpu.sync_copy(thr_hbm, thr_v)

    @pl.loop(0, _HMAX)
    def _(k):
        h = wid + _NW * k

        @pl.when(h < _HIST)
        def _():
            pltpu.async_copy(ts_hbm.at[h], ts_v, sem_i).wait()
            @pl.loop(0, _BATCH, step=16)
            def _(v0):
                tsv = ts_v[pl.ds(v0, 16)]
                acc = jnp.zeros((16,), jnp.int32)
                for i in range(_NTHR):
                    acc = acc + jnp.where(tsv >= thr_v[i, :], 1, 0
                                          ).astype(jnp.int32)
                id_v[pl.ds(v0, 16)] = acc

            @pl.loop(0, _NCH // 2)
            def _(j):
                ch0 = 2 * j

                @pl.when(j > 0)
                def _():
                    pltpu.make_async_copy(
                        st_a, out_hbm.at[h, :, pl.ds(0, _CW)], sem_a).wait()
                _fill_stage(tab_v, id_v, st_a, ch0 * _CW)
                pltpu.async_copy(
                    st_a, out_hbm.at[h, :, pl.ds(ch0 * _CW, _CW)], sem_a)

                @pl.when(j > 0)
                def _():
                    pltpu.make_async_copy(
                        st_b, out_hbm.at[h, :, pl.ds(0, _CW)], sem_b).wait()
                _fill_stage(tab_v, id_v, st_b, (ch0 + 1) * _CW)
                pltpu.async_copy(
                    st_b, out_hbm.at[h, :, pl.ds((ch0 + 1) * _CW, _CW)], sem_b)

            pltpu.make_async_copy(
                st_a, out_hbm.at[h, :, pl.ds(0, _CW)], sem_a).wait()
            pltpu.make_async_copy(
                st_b, out_hbm.at[h, :, pl.ds(0, _CW)], sem_b).wait()


def kernel(timestamps, time_emb_weight):
    boundaries = jnp.concatenate(
        [jnp.zeros((1,), jnp.float32),
         jnp.exp(jnp.arange(_BUCKET_LEN, dtype=jnp.float32))], axis=0)
    rel = boundaries[:_NTHR]
    m0 = jnp.floor(rel * 3600000.0).astype(jnp.int32)
    cand = m0[:, None] + jnp.arange(-1024, 1025, dtype=jnp.int32)[None, :]
    ok = cand.astype(jnp.float32) / 3600000.0 > rel[:, None]
    m = jnp.min(jnp.where(ok, cand, jnp.int32(2**31 - 1)), axis=1)
    thr = jnp.broadcast_to(m[:, None], (_NTHR, 16))

    tab_flat = jnp.zeros((_OUT_DIM, 16), jnp.float32).at[:, :14].set(
        time_emb_weight.T).reshape(-1)
    ts_t = timestamps.T

    mesh = plsc.VectorSubcoreMesh(core_axis_name="c", subcore_axis_name="s")
    cp = pltpu.CompilerParams()
    if "needs_layout_passes" in pltpu.CompilerParams.__dataclass_fields__:
        cp = dataclasses.replace(cp, needs_layout_passes=False)
    run = pl.kernel(
        _sc_body,
        out_type=jax.ShapeDtypeStruct((_HIST, _OUT_DIM, _BATCH), jnp.float32),
        mesh=mesh,
        compiler_params=cp,
        scratch_types=[
            pltpu.VMEM((_OUT_DIM * 16,), jnp.float32),
            pltpu.VMEM((_NTHR, 16), jnp.int32),
            pltpu.VMEM((_BATCH,), jnp.int32),
            pltpu.VMEM((_BATCH,), jnp.int32),
            pltpu.VMEM((_OUT_DIM, _CW), jnp.float32),
            pltpu.VMEM((_OUT_DIM, _CW), jnp.float32),
            pltpu.SemaphoreType.DMA,
            pltpu.SemaphoreType.DMA,
            pltpu.SemaphoreType.DMA,
        ],
    )
    out_t = run(tab_flat, thr, ts_t)
    return jnp.transpose(out_t, (2, 0, 1))


# speedup vs baseline: 1.8871x; 1.0284x over previous
import dataclasses

import jax
import jax.numpy as jnp
from jax import lax
from jax.experimental import pallas as pl
from jax.experimental.pallas import tpu as pltpu
from jax.experimental.pallas import tpu_sc as plsc

_BATCH = 4096
_HIST = 200
_BUCKET_LEN = 12
_OUT_DIM = 72
_NTHR = 8
_CW = 512
_NCH = _BATCH // _CW
_NW = 32
_HMAX = (_HIST + _NW - 1) // _NW


def _fill_stage(tab_v, id_v, st, base):
    @plsc.parallel_loop(0, _CW, 16, unroll=8)
    def _(v0):
        idb = id_v[pl.ds(base + v0, 16)]
        depth = 16
        pend = []
        for c in range(_OUT_DIM):
            pend.append(
                (c, plsc.load_gather(tab_v.at[pl.ds(c * 16, 16)], [idb])))
            if len(pend) >= depth:
                cc, val = pend.pop(0)
                st[cc, pl.ds(v0, 16)] = val
        for cc, val in pend:
            st[cc, pl.ds(v0, 16)] = val


def _sc_body(tab_hbm, thr_hbm, ts_hbm, out_hbm,
             tab_v, thr_v, ts_v, id_v, st_a, st_b,
             sem_i, sem_a, sem_b):
    wid = lax.axis_index("s") * 2 + lax.axis_index("c")
    pltpu.sync_copy(tab_hbm, tab_v)
    pltpu.sync_copy(thr_hbm, thr_v)

    @pl.loop(0, _HMAX)
    def _(k):
        h = wid + _NW * k

        @pl.when(h < _HIST)
        def _():
            pltpu.async_copy(ts_hbm.at[h], ts_v, sem_i).wait()
            @pl.loop(0, _BATCH, step=16)
            def _(v0):
                tsv = ts_v[pl.ds(v0, 16)]
                acc = jnp.zeros((16,), jnp.int32)
                for i in range(_NTHR):
                    acc = acc + jnp.where(tsv >= thr_v[i, :], 1, 0
                                          ).astype(jnp.int32)
                id_v[pl.ds(v0, 16)] = acc

            @pl.loop(0, _NCH // 2)
            def _(j):
                ch0 = 2 * j

                @pl.when(j > 0)
                def _():
                    pltpu.make_async_copy(
                        st_a, out_hbm.at[h, :, pl.ds(0, _CW)], sem_a).wait()
                _fill_stage(tab_v, id_v, st_a, ch0 * _CW)
                pltpu.async_copy(
                    st_a, out_hbm.at[h, :, pl.ds(ch0 * _CW, _CW)], sem_a)

                @pl.when(j > 0)
                def _():
                    pltpu.make_async_copy(
                        st_b, out_hbm.at[h, :, pl.ds(0, _CW)], sem_b).wait()
                _fill_stage(tab_v, id_v, st_b, (ch0 + 1) * _CW)
                pltpu.async_copy(
                    st_b, out_hbm.at[h, :, pl.ds((ch0 + 1) * _CW, _CW)], sem_b)

            pltpu.make_async_copy(
                st_a, out_hbm.at[h, :, pl.ds(0, _CW)], sem_a).wait()
            pltpu.make_async_copy(
                st_b, out_hbm.at[h, :, pl.ds(0, _CW)], sem_b).wait()


def kernel(timestamps, time_emb_weight):
    boundaries = jnp.concatenate(
        [jnp.zeros((1,), jnp.float32),
         jnp.exp(jnp.arange(_BUCKET_LEN, dtype=jnp.float32))], axis=0)
    rel = boundaries[:_NTHR]
    m0 = jnp.floor(rel * 3600000.0).astype(jnp.int32)
    cand = m0[:, None] + jnp.arange(-1024, 1025, dtype=jnp.int32)[None, :]
    ok = cand.astype(jnp.float32) / 3600000.0 > rel[:, None]
    m = jnp.min(jnp.where(ok, cand, jnp.int32(2**31 - 1)), axis=1)
    thr = jnp.broadcast_to(m[:, None], (_NTHR, 16))

    tab_flat = jnp.zeros((_OUT_DIM, 16), jnp.float32).at[:, :14].set(
        time_emb_weight.T).reshape(-1)
    ts_t = timestamps.T

    mesh = plsc.VectorSubcoreMesh(core_axis_name="c", subcore_axis_name="s")
    cp = pltpu.CompilerParams()
    if "needs_layout_passes" in pltpu.CompilerParams.__dataclass_fields__:
        cp = dataclasses.replace(cp, needs_layout_passes=False)
    run = pl.kernel(
        _sc_body,
        out_type=jax.ShapeDtypeStruct((_HIST, _OUT_DIM, _BATCH), jnp.float32),
        mesh=mesh,
        compiler_params=cp,
        scratch_types=[
            pltpu.VMEM((_OUT_DIM * 16,), jnp.float32),
            pltpu.VMEM((_NTHR, 16), jnp.int32),
            pltpu.VMEM((_BATCH,), jnp.int32),
            pltpu.VMEM((_BATCH,), jnp.int32),
            pltpu.VMEM((_OUT_DIM, _CW), jnp.float32),
            pltpu.VMEM((_OUT_DIM, _CW), jnp.float32),
            pltpu.SemaphoreType.DMA,
            pltpu.SemaphoreType.DMA,
            pltpu.SemaphoreType.DMA,
        ],
    )
    out_t = run(tab_flat, thr, ts_t)
    return jnp.transpose(out_t, (2, 0, 1))
